# baseline (speedup 1.0000x reference)
import functools
import math

import jax
import jax.numpy as jnp
from jax import lax
from jax.experimental import pallas as pl
from jax.experimental.pallas import tpu as pltpu
from jax.experimental.pallas import tpu_sc as plsc

D_MODEL = 128
BATCH = 4096
SEQ = 50
COEFF = math.sqrt(float(D_MODEL))

NUM_CORES = 2
NUM_SUBCORES = 16
NW = NUM_CORES * NUM_SUBCORES
BPW = BATCH // NW
EPC = 2
ROWS = EPC * SEQ
NCHUNK = BPW // EPC
NBUF = 8
K = NBUF // 2

_mesh = plsc.VectorSubcoreMesh(core_axis_name="c", subcore_axis_name="s")


@functools.partial(
    pl.kernel,
    mesh=_mesh,
    out_type=jax.ShapeDtypeStruct((BATCH, SEQ, D_MODEL), jnp.float32),
    scratch_types=(
        [pltpu.VMEM((BPW, SEQ), jnp.int32)]
        + [pltpu.VMEM((EPC, SEQ, D_MODEL), jnp.float32)] * NBUF
        + [pltpu.SemaphoreType.DMA] * (2 * NBUF)
    ),
)
def _emb_lookup(idx_hbm, table_hbm, out_hbm, idx_v, *rest):
    bufs = rest[:NBUF]
    gsems = rest[NBUF:2 * NBUF]
    ssems = rest[2 * NBUF:]

    cid = lax.axis_index("c")
    sid = lax.axis_index("s")
    wid = sid * NUM_CORES + cid
    base = wid * BPW

    pltpu.sync_copy(idx_hbm.at[wid], idx_v)

    def gather_start(g, b):
        for h in range(EPC):
            pltpu.make_async_copy(
                table_hbm.at[idx_v.at[g * EPC + h]],
                bufs[b].at[h], gsems[b]).start()

    def gather_wait(b):
        for h in range(EPC):
            pltpu.make_async_copy(
                table_hbm.at[idx_v.at[0]], bufs[b].at[h], gsems[b]).wait()

    def store_start(g, b):
        pltpu.make_async_copy(
            bufs[b], out_hbm.at[pl.ds(base + g * EPC, EPC)],
            ssems[b]).start()

    def store_wait(b):
        pltpu.make_async_copy(
            bufs[b], out_hbm.at[pl.ds(base, EPC)], ssems[b]).wait()

    def scale(buf):
        for e in range(EPC):
            @plsc.parallel_loop(0, SEQ, step=2, unroll=2)
            def _rows(r):
                for rr in range(2):
                    for c in range(D_MODEL // 16):
                        sl = pl.ds(c * 16, 16)
                        buf[e, r + rr, sl] = buf[e, r + rr, sl] * COEFF

    for c in range(K):
        gather_start(c, c)

    def outer(gg, carry):
        for b in range(NBUF):
            g = gg * NBUF + b
            tb = (b + K) % NBUF

            @pl.when(g >= NBUF - K)
            def _():
                store_wait(tb)

            @pl.when(g + K < NCHUNK)
            def _():
                gather_start(g + K, tb)

            gather_wait(b)
            scale(bufs[b])
            store_start(g, b)
        return carry

    lax.fori_loop(0, NCHUNK // NBUF, outer, 0)
    for c in range(NCHUNK - NBUF + K, NCHUNK):
        store_wait(c % NBUF)


def kernel(x, table):
    idx = x.reshape(NW, BPW, SEQ).astype(jnp.int32)
    return _emb_lookup(idx, table.astype(jnp.float32))


# speedup vs baseline: 1.0266x; 1.0266x over previous
import functools
import math

import jax
import jax.numpy as jnp
from jax import lax
from jax.experimental import pallas as pl
from jax.experimental.pallas import tpu as pltpu
from jax.experimental.pallas import tpu_sc as plsc

D_MODEL = 128
BATCH = 4096
SEQ = 50
COEFF = math.sqrt(float(D_MODEL))

NUM_CORES = 2
NUM_SUBCORES = 16
NW = NUM_CORES * NUM_SUBCORES
BPW = BATCH // NW
EPC = 2
ROWS = EPC * SEQ
NCHUNK = BPW // EPC
NBUF = 8
K = 6

_mesh = plsc.VectorSubcoreMesh(core_axis_name="c", subcore_axis_name="s")


@functools.partial(
    pl.kernel,
    mesh=_mesh,
    out_type=jax.ShapeDtypeStruct((BATCH, SEQ, D_MODEL), jnp.float32),
    scratch_types=(
        [pltpu.VMEM((NCHUNK, ROWS), jnp.int32)]
        + [pltpu.VMEM((ROWS, D_MODEL), jnp.float32)] * NBUF
        + [pltpu.SemaphoreType.DMA] * (2 * NBUF)
    ),
)
def _emb_lookup(idx_hbm, table_hbm, out_hbm, idx_v, *rest):
    bufs = rest[:NBUF]
    gsems = rest[NBUF:2 * NBUF]
    ssems = rest[2 * NBUF:]

    cid = lax.axis_index("c")
    sid = lax.axis_index("s")
    wid = sid * NUM_CORES + cid
    base = wid * BPW

    pltpu.sync_copy(idx_hbm.at[wid], idx_v)

    def gather_start(g, b):
        pltpu.make_async_copy(
            table_hbm.at[idx_v.at[g]], bufs[b], gsems[b]).start()

    def gather_wait(b):
        pltpu.make_async_copy(
            table_hbm.at[idx_v.at[0]], bufs[b], gsems[b]).wait()

    def store_start(g, b):
        for h in range(EPC):
            pltpu.make_async_copy(
                bufs[b].at[pl.ds(h * SEQ, SEQ)],
                out_hbm.at[base + g * EPC + h], ssems[b]).start()

    def store_wait(b):
        for h in range(EPC):
            pltpu.make_async_copy(
                bufs[b].at[pl.ds(h * SEQ, SEQ)],
                out_hbm.at[base], ssems[b]).wait()

    def scale(buf):
        @plsc.parallel_loop(0, ROWS, step=2, unroll=2)
        def _rows(r):
            for rr in range(2):
                for c in range(D_MODEL // 16):
                    sl = pl.ds(c * 16, 16)
                    buf[r + rr, sl] = buf[r + rr, sl] * COEFF

    for c in range(K):
        gather_start(c, c)

    def outer(gg, carry):
        for b in range(NBUF):
            g = gg * NBUF + b
            tb = (b + K) % NBUF

            @pl.when(g >= NBUF - K)
            def _():
                store_wait(tb)

            @pl.when(g + K < NCHUNK)
            def _():
                gather_start(g + K, tb)

            gather_wait(b)
            scale(bufs[b])
            store_start(g, b)
        return carry

    lax.fori_loop(0, NCHUNK // NBUF, outer, 0)
    for c in range(NCHUNK - NBUF + K, NCHUNK):
        store_wait(c % NBUF)


def kernel(x, table):
    idx = x.reshape(NW, NCHUNK, ROWS).astype(jnp.int32)
    return _emb_lookup(idx, table.astype(jnp.float32))
